# baseline (speedup 1.0000x reference)
import functools

import jax
import jax.numpy as jnp
from jax import lax
from jax.experimental import pallas as pl
from jax.experimental.pallas import tpu as pltpu
from jax.experimental.pallas import tpu_sc as plsc

N = 10000
E = 320000
D = 128
G = 256

NC = 2
NS = 16
NW = NC * NS

CHUNK = 128
CPW = 80
E_PAD = NW * CPW * CHUNK
N_PAD = 10240
ACC_ROWS = 10112
ROWS_PER_TILE = ACC_ROWS // NS

BN = 1024
NB = N_PAD // BN


NBUF = 2


def _sc_agg_body(h_hbm, idx_hbm, zeros_hbm, out_hbm,
                 idx_v, src_u, dst_u, rows, acc, gs, ss):
    cc = lax.axis_index("c")
    s = lax.axis_index("s")
    wid = cc * NS + s

    r0 = s * ROWS_PER_TILE
    pltpu.sync_copy(zeros_hbm.at[pl.ds(r0, ROWS_PER_TILE)],
                    acc.at[pl.ds(r0, ROWS_PER_TILE)])

    pltpu.sync_copy(idx_hbm.at[wid], idx_v)

    plsc.subcore_barrier()

    def unpack(ch, i):
        for q in range(CHUNK // 16):
            v = idx_v[ch, pl.ds(q * 16, 16)]
            src_u[i, pl.ds(q * 16, 16)] = v & 0xFFFF
            dst_u[i, pl.ds(q * 16, 16)] = lax.shift_right_logical(v, 16)

    def gather(ch, i):
        pltpu.async_copy(h_hbm.at[src_u.at[i]], rows.at[i], gs.at[i])

    def gather_wait(i):
        pltpu.make_async_copy(h_hbm.at[src_u.at[i]], rows.at[i],
                              gs.at[i]).wait()

    def scat(i):
        pltpu.async_copy(rows.at[i], acc.at[dst_u.at[i]], ss.at[i], add=True)

    def scat_wait(i):
        pltpu.make_async_copy(rows.at[i], acc.at[dst_u.at[i]],
                              ss.at[i]).wait()

    unpack(0, 0)
    unpack(1, 1)
    gather(0, 0)
    gather(1, 1)

    @pl.loop(0, CPW, step=NBUF)
    def _(k):
        for i in range(NBUF):
            c = k + i
            o = 1 - i
            gather_wait(i)
            scat(i)

            @pl.when(c >= 1)
            def _():
                scat_wait(o)

                @pl.when(c + 1 < CPW)
                def _():
                    unpack(c + 1, o)
                    gather(c + 1, o)

    scat_wait((CPW - 1) % NBUF)

    plsc.subcore_barrier()

    pltpu.sync_copy(acc.at[pl.ds(r0, ROWS_PER_TILE)],
                    out_hbm.at[cc, pl.ds(r0, ROWS_PER_TILE)])


_sc_agg = functools.partial(
    pl.kernel,
    out_type=jax.ShapeDtypeStruct((NC, N_PAD, D), jnp.float32),
    mesh=plsc.VectorSubcoreMesh(core_axis_name="c", subcore_axis_name="s"),
    scratch_types=[
        pltpu.VMEM((CPW, CHUNK), jnp.int32),
        pltpu.VMEM((NBUF, CHUNK), jnp.int32),
        pltpu.VMEM((NBUF, CHUNK), jnp.int32),
        pltpu.VMEM((NBUF, CHUNK, D), jnp.float32),
        pltpu.VMEM_SHARED((ACC_ROWS, D), jnp.float32),
        pltpu.SemaphoreType.DMA((NBUF,)),
        pltpu.SemaphoreType.DMA((NBUF,)),
    ],
)(_sc_agg_body)


def _dense_body(p_ref, h_ref, w_ref, r_ref, b_ref, rb_ref, o_ref):
    agg = p_ref[0] + p_ref[1]
    conv = jnp.maximum(
        jnp.dot(agg, w_ref[...], preferred_element_type=jnp.float32,
                precision=lax.Precision.HIGHEST)
        + b_ref[...], 0.0)
    res = jnp.maximum(
        jnp.dot(h_ref[...], r_ref[...], preferred_element_type=jnp.float32,
                precision=lax.Precision.HIGHEST)
        + rb_ref[...], 0.0)
    o_ref[...] = conv + res


_dense = pl.pallas_call(
    _dense_body,
    grid=(NB,),
    in_specs=[
        pl.BlockSpec((2, BN, D), lambda i: (0, i, 0)),
        pl.BlockSpec((BN, D), lambda i: (i, 0)),
        pl.BlockSpec((D, D), lambda i: (0, 0)),
        pl.BlockSpec((D, D), lambda i: (0, 0)),
        pl.BlockSpec((1, D), lambda i: (0, 0)),
        pl.BlockSpec((1, D), lambda i: (0, 0)),
    ],
    out_specs=pl.BlockSpec((BN, D), lambda i: (i, 0)),
    out_shape=jax.ShapeDtypeStruct((N_PAD, D), jnp.float32),
)


def _dense_pool_body(p_ref, h_ref, w_ref, r_ref, b_ref, rb_ref, n2g_ref,
                     o_ref, g_ref):
    agg = p_ref[0] + p_ref[1]
    conv = jnp.maximum(
        jnp.dot(agg, w_ref[...], preferred_element_type=jnp.float32,
                precision=lax.Precision.HIGHEST)
        + b_ref[...], 0.0)
    res = jnp.maximum(
        jnp.dot(h_ref[...], r_ref[...], preferred_element_type=jnp.float32,
                precision=lax.Precision.HIGHEST)
        + rb_ref[...], 0.0)
    hn = conv + res
    o_ref[...] = hn

    n2g = n2g_ref[0, 0]
    hn = jnp.where(n2g[:, None] < G, hn, 0.0)
    onehot = (n2g[:, None]
              == lax.broadcasted_iota(jnp.int32, (BN, G), 1)).astype(jnp.float32)
    gpart = lax.dot_general(onehot, hn, (((0,), (0,)), ((), ())),
                            preferred_element_type=jnp.float32,
                precision=lax.Precision.HIGHEST)

    @pl.when(pl.program_id(0) == 0)
    def _():
        g_ref[...] = jnp.zeros_like(g_ref)

    g_ref[...] += gpart


_dense_pool = pl.pallas_call(
    _dense_pool_body,
    grid=(NB,),
    in_specs=[
        pl.BlockSpec((2, BN, D), lambda i: (0, i, 0)),
        pl.BlockSpec((BN, D), lambda i: (i, 0)),
        pl.BlockSpec((D, D), lambda i: (0, 0)),
        pl.BlockSpec((D, D), lambda i: (0, 0)),
        pl.BlockSpec((1, D), lambda i: (0, 0)),
        pl.BlockSpec((1, D), lambda i: (0, 0)),
        pl.BlockSpec((1, 1, BN), lambda i: (i, 0, 0)),
    ],
    out_specs=[
        pl.BlockSpec((BN, D), lambda i: (i, 0)),
        pl.BlockSpec((G, D), lambda i: (0, 0)),
    ],
    out_shape=[
        jax.ShapeDtypeStruct((N_PAD, D), jnp.float32),
        jax.ShapeDtypeStruct((G, D), jnp.float32),
    ],
)


def _mlp_body(g_ref, w1_ref, b1_ref, w2_ref, b2_ref, o_ref):
    hm = jnp.maximum(
        jnp.dot(g_ref[...], w1_ref[...], preferred_element_type=jnp.float32,
                precision=lax.Precision.HIGHEST)
        + b1_ref[...], 0.0)
    o_ref[...] = (jnp.dot(hm, w2_ref[...], preferred_element_type=jnp.float32,
                precision=lax.Precision.HIGHEST)
                  + b2_ref[...])


_mlp = pl.pallas_call(
    _mlp_body,
    out_shape=jax.ShapeDtypeStruct((G, 1), jnp.float32),
)


@jax.jit
def kernel(graph_feats, edge_index, node2graph,
           W1, b1, R1, rb1, W2, b2, R2, rb2, W3, b3, R3, rb3,
           Wm1, bm1, Wm2, bm2):
    src = edge_index[0]
    dst = edge_index[1]
    pad = E_PAD - E
    src_p = jnp.concatenate([src, jnp.zeros((pad,), jnp.int32)])
    dst_p = jnp.concatenate([dst, jnp.full((pad,), N, jnp.int32)])
    idx_r = (src_p | (dst_p << 16)).reshape(NW, CPW, CHUNK)

    h = jnp.pad(graph_feats, ((0, N_PAD - N), (0, 0)))
    zeros = jnp.zeros((N_PAD, D), jnp.float32)
    n2g3 = jnp.pad(node2graph, (0, N_PAD - N),
                   constant_values=G).reshape(NB, 1, BN)

    for (W, b, Rw, rb) in ((W1, b1, R1, rb1), (W2, b2, R2, rb2)):
        parts = _sc_agg(h, idx_r, zeros)
        h = _dense(parts, h, W, Rw, b.reshape(1, D), rb.reshape(1, D))

    parts = _sc_agg(h, idx_r, zeros)
    h, g = _dense_pool(parts, h, W3, R3, b3.reshape(1, D), rb3.reshape(1, D),
                       n2g3)

    return _mlp(g, Wm1, bm1.reshape(1, D), Wm2, bm2.reshape(1, 1))


# speedup vs baseline: 1.0257x; 1.0257x over previous
import functools

import jax
import jax.numpy as jnp
from jax import lax
from jax.experimental import pallas as pl
from jax.experimental.pallas import tpu as pltpu
from jax.experimental.pallas import tpu_sc as plsc

N = 10000
E = 320000
D = 128
G = 256

NC = 2
NS = 16
NW = NC * NS

CHUNK = 128
CPW = 80
E_PAD = NW * CPW * CHUNK
N_PAD = 10240
ACC_ROWS = 10112
ROWS_PER_TILE = ACC_ROWS // NS

BN = 1024
NB = N_PAD // BN


NBUF = 2


def _sc_agg_body(h_hbm, src_hbm, dst_hbm, zeros_hbm, out_hbm,
                 src_v, dst_v, rows, acc, gs, ss):
    cc = lax.axis_index("c")
    s = lax.axis_index("s")
    wid = cc * NS + s

    r0 = s * ROWS_PER_TILE
    pltpu.sync_copy(zeros_hbm.at[pl.ds(r0, ROWS_PER_TILE)],
                    acc.at[pl.ds(r0, ROWS_PER_TILE)])

    pltpu.sync_copy(src_hbm.at[wid], src_v)
    pltpu.sync_copy(dst_hbm.at[wid], dst_v)

    plsc.subcore_barrier()

    @pl.loop(0, CPW, step=1)
    def _(c):
        pltpu.async_copy(h_hbm.at[src_v.at[c]], rows.at[0], gs.at[0])
        pltpu.make_async_copy(h_hbm.at[src_v.at[c]], rows.at[0],
                              gs.at[0]).wait()
        pltpu.async_copy(rows.at[0], acc.at[dst_v.at[c]], ss.at[0], add=True)
        pltpu.make_async_copy(rows.at[0], acc.at[dst_v.at[c]],
                              ss.at[0]).wait()

    plsc.subcore_barrier()

    pltpu.sync_copy(acc.at[pl.ds(r0, ROWS_PER_TILE)],
                    out_hbm.at[cc, pl.ds(r0, ROWS_PER_TILE)])


_sc_agg = functools.partial(
    pl.kernel,
    out_type=jax.ShapeDtypeStruct((NC, N_PAD, D), jnp.float32),
    mesh=plsc.VectorSubcoreMesh(core_axis_name="c", subcore_axis_name="s"),
    scratch_types=[
        pltpu.VMEM((CPW, CHUNK), jnp.int32),
        pltpu.VMEM((CPW, CHUNK), jnp.int32),
        pltpu.VMEM((1, CHUNK, D), jnp.float32),
        pltpu.VMEM_SHARED((ACC_ROWS, D), jnp.float32),
        pltpu.SemaphoreType.DMA((1,)),
        pltpu.SemaphoreType.DMA((1,)),
    ],
)(_sc_agg_body)


def _dense_body(p_ref, h_ref, w_ref, r_ref, b_ref, rb_ref, o_ref):
    agg = p_ref[0] + p_ref[1]
    conv = jnp.maximum(
        jnp.dot(agg, w_ref[...], preferred_element_type=jnp.float32,
                precision=lax.Precision.HIGHEST)
        + b_ref[...], 0.0)
    res = jnp.maximum(
        jnp.dot(h_ref[...], r_ref[...], preferred_element_type=jnp.float32,
                precision=lax.Precision.HIGHEST)
        + rb_ref[...], 0.0)
    o_ref[...] = conv + res


_dense = pl.pallas_call(
    _dense_body,
    grid=(NB,),
    in_specs=[
        pl.BlockSpec((2, BN, D), lambda i: (0, i, 0)),
        pl.BlockSpec((BN, D), lambda i: (i, 0)),
        pl.BlockSpec((D, D), lambda i: (0, 0)),
        pl.BlockSpec((D, D), lambda i: (0, 0)),
        pl.BlockSpec((1, D), lambda i: (0, 0)),
        pl.BlockSpec((1, D), lambda i: (0, 0)),
    ],
    out_specs=pl.BlockSpec((BN, D), lambda i: (i, 0)),
    out_shape=jax.ShapeDtypeStruct((N_PAD, D), jnp.float32),
)


def _dense_pool_body(p_ref, h_ref, w_ref, r_ref, b_ref, rb_ref, n2g_ref,
                     o_ref, g_ref):
    agg = p_ref[0] + p_ref[1]
    conv = jnp.maximum(
        jnp.dot(agg, w_ref[...], preferred_element_type=jnp.float32,
                precision=lax.Precision.HIGHEST)
        + b_ref[...], 0.0)
    res = jnp.maximum(
        jnp.dot(h_ref[...], r_ref[...], preferred_element_type=jnp.float32,
                precision=lax.Precision.HIGHEST)
        + rb_ref[...], 0.0)
    hn = conv + res
    o_ref[...] = hn

    n2g = n2g_ref[0, 0]
    hn = jnp.where(n2g[:, None] < G, hn, 0.0)
    onehot = (n2g[:, None]
              == lax.broadcasted_iota(jnp.int32, (BN, G), 1)).astype(jnp.float32)
    gpart = lax.dot_general(onehot, hn, (((0,), (0,)), ((), ())),
                            preferred_element_type=jnp.float32,
                precision=lax.Precision.HIGHEST)

    @pl.when(pl.program_id(0) == 0)
    def _():
        g_ref[...] = jnp.zeros_like(g_ref)

    g_ref[...] += gpart


_dense_pool = pl.pallas_call(
    _dense_pool_body,
    grid=(NB,),
    in_specs=[
        pl.BlockSpec((2, BN, D), lambda i: (0, i, 0)),
        pl.BlockSpec((BN, D), lambda i: (i, 0)),
        pl.BlockSpec((D, D), lambda i: (0, 0)),
        pl.BlockSpec((D, D), lambda i: (0, 0)),
        pl.BlockSpec((1, D), lambda i: (0, 0)),
        pl.BlockSpec((1, D), lambda i: (0, 0)),
        pl.BlockSpec((1, 1, BN), lambda i: (i, 0, 0)),
    ],
    out_specs=[
        pl.BlockSpec((BN, D), lambda i: (i, 0)),
        pl.BlockSpec((G, D), lambda i: (0, 0)),
    ],
    out_shape=[
        jax.ShapeDtypeStruct((N_PAD, D), jnp.float32),
        jax.ShapeDtypeStruct((G, D), jnp.float32),
    ],
)


def _mlp_body(g_ref, w1_ref, b1_ref, w2_ref, b2_ref, o_ref):
    hm = jnp.maximum(
        jnp.dot(g_ref[...], w1_ref[...], preferred_element_type=jnp.float32,
                precision=lax.Precision.HIGHEST)
        + b1_ref[...], 0.0)
    o_ref[...] = (jnp.dot(hm, w2_ref[...], preferred_element_type=jnp.float32,
                precision=lax.Precision.HIGHEST)
                  + b2_ref[...])


_mlp = pl.pallas_call(
    _mlp_body,
    out_shape=jax.ShapeDtypeStruct((G, 1), jnp.float32),
)


@jax.jit
def kernel(graph_feats, edge_index, node2graph,
           W1, b1, R1, rb1, W2, b2, R2, rb2, W3, b3, R3, rb3,
           Wm1, bm1, Wm2, bm2):
    src = edge_index[0]
    dst = edge_index[1]
    pad = E_PAD - E
    src_p = jnp.concatenate([src, jnp.zeros((pad,), jnp.int32)])
    dst_p = jnp.concatenate([dst, jnp.full((pad,), N, jnp.int32)])
    src_r = src_p.reshape(NW, CPW, CHUNK)
    dst_r = dst_p.reshape(NW, CPW, CHUNK)

    h = jnp.pad(graph_feats, ((0, N_PAD - N), (0, 0)))
    zeros = jnp.zeros((N_PAD, D), jnp.float32)
    n2g3 = jnp.pad(node2graph, (0, N_PAD - N),
                   constant_values=G).reshape(NB, 1, BN)

    for (W, b, Rw, rb) in ((W1, b1, R1, rb1), (W2, b2, R2, rb2)):
        parts = _sc_agg(h, src_r, dst_r, zeros)
        h = _dense(parts, h, W, Rw, b.reshape(1, D), rb.reshape(1, D))

    parts = _sc_agg(h, src_r, dst_r, zeros)
    h, g = _dense_pool(parts, h, W3, R3, b3.reshape(1, D), rb3.reshape(1, D),
                       n2g3)

    return _mlp(g, Wm1, bm1.reshape(1, D), Wm2, bm2.reshape(1, 1))


# speedup vs baseline: 1.1910x; 1.1612x over previous
import functools

import jax
import jax.numpy as jnp
from jax import lax
from jax.experimental import pallas as pl
from jax.experimental.pallas import tpu as pltpu
from jax.experimental.pallas import tpu_sc as plsc

N = 10000
E = 320000
D = 128
G = 256

NC = 2
NS = 16
NW = NC * NS

CHUNK = 128
NCHUNKS = 2560
E_PAD = NCHUNKS * CHUNK
CPW0 = 128
CPW1 = 32
N_PAD = 10240
ACC_ROWS = 10112
ROWS_PER_TILE = ACC_ROWS // NS

BN = 1024
NB = N_PAD // BN


NBUF = 2


def _sc_agg_body(h_hbm, src_hbm, dst_hbm, zeros_hbm, out_hbm,
                 src_v, dst_v, rows, acc, gs, ss):
    cc = lax.axis_index("c")
    s = lax.axis_index("s")

    r0 = s * ROWS_PER_TILE
    pltpu.sync_copy(zeros_hbm.at[pl.ds(r0, ROWS_PER_TILE)],
                    acc.at[pl.ds(r0, ROWS_PER_TILE)])

    @pl.when(cc == 0)
    def _():
        base = s * CPW0
        pltpu.sync_copy(src_hbm.at[pl.ds(base, CPW0)], src_v)
        pltpu.sync_copy(dst_hbm.at[pl.ds(base, CPW0)], dst_v)

    @pl.when(cc == 1)
    def _():
        base = NS * CPW0 + s * CPW1
        pltpu.sync_copy(src_hbm.at[pl.ds(base, CPW1)],
                        src_v.at[pl.ds(0, CPW1)])
        pltpu.sync_copy(dst_hbm.at[pl.ds(base, CPW1)],
                        dst_v.at[pl.ds(0, CPW1)])

    plsc.subcore_barrier()

    def step(c):
        pltpu.async_copy(h_hbm.at[src_v.at[c]], rows.at[0], gs.at[0])
        pltpu.make_async_copy(h_hbm.at[src_v.at[c]], rows.at[0],
                              gs.at[0]).wait()
        pltpu.async_copy(rows.at[0], acc.at[dst_v.at[c]], ss.at[0], add=True)
        pltpu.make_async_copy(rows.at[0], acc.at[dst_v.at[c]],
                              ss.at[0]).wait()

    @pl.when(cc == 0)
    def _():
        @pl.loop(0, CPW0, step=1)
        def _(c):
            step(c)

    @pl.when(cc == 1)
    def _():
        @pl.loop(0, CPW1, step=1)
        def _(c):
            step(c)

    plsc.subcore_barrier()

    pltpu.sync_copy(acc.at[pl.ds(r0, ROWS_PER_TILE)],
                    out_hbm.at[cc, pl.ds(r0, ROWS_PER_TILE)])


_sc_agg = functools.partial(
    pl.kernel,
    out_type=jax.ShapeDtypeStruct((NC, N_PAD, D), jnp.float32),
    mesh=plsc.VectorSubcoreMesh(core_axis_name="c", subcore_axis_name="s"),
    scratch_types=[
        pltpu.VMEM((CPW0, CHUNK), jnp.int32),
        pltpu.VMEM((CPW0, CHUNK), jnp.int32),
        pltpu.VMEM((1, CHUNK, D), jnp.float32),
        pltpu.VMEM_SHARED((ACC_ROWS, D), jnp.float32),
        pltpu.SemaphoreType.DMA((1,)),
        pltpu.SemaphoreType.DMA((1,)),
    ],
)(_sc_agg_body)


def _dense_body(p_ref, h_ref, w_ref, r_ref, b_ref, rb_ref, o_ref):
    agg = p_ref[0] + p_ref[1]
    conv = jnp.maximum(
        jnp.dot(agg, w_ref[...], preferred_element_type=jnp.float32,
                precision=lax.Precision.HIGHEST)
        + b_ref[...], 0.0)
    res = jnp.maximum(
        jnp.dot(h_ref[...], r_ref[...], preferred_element_type=jnp.float32,
                precision=lax.Precision.HIGHEST)
        + rb_ref[...], 0.0)
    o_ref[...] = conv + res


_dense = pl.pallas_call(
    _dense_body,
    grid=(NB,),
    in_specs=[
        pl.BlockSpec((2, BN, D), lambda i: (0, i, 0)),
        pl.BlockSpec((BN, D), lambda i: (i, 0)),
        pl.BlockSpec((D, D), lambda i: (0, 0)),
        pl.BlockSpec((D, D), lambda i: (0, 0)),
        pl.BlockSpec((1, D), lambda i: (0, 0)),
        pl.BlockSpec((1, D), lambda i: (0, 0)),
    ],
    out_specs=pl.BlockSpec((BN, D), lambda i: (i, 0)),
    out_shape=jax.ShapeDtypeStruct((N_PAD, D), jnp.float32),
)


def _dense_pool_body(p_ref, h_ref, w_ref, r_ref, b_ref, rb_ref, n2g_ref,
                     o_ref, g_ref):
    agg = p_ref[0] + p_ref[1]
    conv = jnp.maximum(
        jnp.dot(agg, w_ref[...], preferred_element_type=jnp.float32,
                precision=lax.Precision.HIGHEST)
        + b_ref[...], 0.0)
    res = jnp.maximum(
        jnp.dot(h_ref[...], r_ref[...], preferred_element_type=jnp.float32,
                precision=lax.Precision.HIGHEST)
        + rb_ref[...], 0.0)
    hn = conv + res
    o_ref[...] = hn

    n2g = n2g_ref[0, 0]
    hn = jnp.where(n2g[:, None] < G, hn, 0.0)
    onehot = (n2g[:, None]
              == lax.broadcasted_iota(jnp.int32, (BN, G), 1)).astype(jnp.float32)
    gpart = lax.dot_general(onehot, hn, (((0,), (0,)), ((), ())),
                            preferred_element_type=jnp.float32,
                precision=lax.Precision.HIGHEST)

    @pl.when(pl.program_id(0) == 0)
    def _():
        g_ref[...] = jnp.zeros_like(g_ref)

    g_ref[...] += gpart


_dense_pool = pl.pallas_call(
    _dense_pool_body,
    grid=(NB,),
    in_specs=[
        pl.BlockSpec((2, BN, D), lambda i: (0, i, 0)),
        pl.BlockSpec((BN, D), lambda i: (i, 0)),
        pl.BlockSpec((D, D), lambda i: (0, 0)),
        pl.BlockSpec((D, D), lambda i: (0, 0)),
        pl.BlockSpec((1, D), lambda i: (0, 0)),
        pl.BlockSpec((1, D), lambda i: (0, 0)),
        pl.BlockSpec((1, 1, BN), lambda i: (i, 0, 0)),
    ],
    out_specs=[
        pl.BlockSpec((BN, D), lambda i: (i, 0)),
        pl.BlockSpec((G, D), lambda i: (0, 0)),
    ],
    out_shape=[
        jax.ShapeDtypeStruct((N_PAD, D), jnp.float32),
        jax.ShapeDtypeStruct((G, D), jnp.float32),
    ],
)


def _mlp_body(g_ref, w1_ref, b1_ref, w2_ref, b2_ref, o_ref):
    hm = jnp.maximum(
        jnp.dot(g_ref[...], w1_ref[...], preferred_element_type=jnp.float32,
                precision=lax.Precision.HIGHEST)
        + b1_ref[...], 0.0)
    o_ref[...] = (jnp.dot(hm, w2_ref[...], preferred_element_type=jnp.float32,
                precision=lax.Precision.HIGHEST)
                  + b2_ref[...])


_mlp = pl.pallas_call(
    _mlp_body,
    out_shape=jax.ShapeDtypeStruct((G, 1), jnp.float32),
)


@jax.jit
def kernel(graph_feats, edge_index, node2graph,
           W1, b1, R1, rb1, W2, b2, R2, rb2, W3, b3, R3, rb3,
           Wm1, bm1, Wm2, bm2):
    src = edge_index[0]
    dst = edge_index[1]
    pad = E_PAD - E
    src_p = jnp.concatenate([src, jnp.zeros((pad,), jnp.int32)])
    dst_p = jnp.concatenate([dst, jnp.full((pad,), N, jnp.int32)])
    src_r = src_p.reshape(NCHUNKS, CHUNK)
    dst_r = dst_p.reshape(NCHUNKS, CHUNK)

    h = jnp.pad(graph_feats, ((0, N_PAD - N), (0, 0)))
    zeros = jnp.zeros((N_PAD, D), jnp.float32)
    n2g3 = jnp.pad(node2graph, (0, N_PAD - N),
                   constant_values=G).reshape(NB, 1, BN)

    for (W, b, Rw, rb) in ((W1, b1, R1, rb1), (W2, b2, R2, rb2)):
        parts = _sc_agg(h, src_r, dst_r, zeros)
        h = _dense(parts, h, W, Rw, b.reshape(1, D), rb.reshape(1, D))

    parts = _sc_agg(h, src_r, dst_r, zeros)
    h, g = _dense_pool(parts, h, W3, R3, b3.reshape(1, D), rb3.reshape(1, D),
                       n2g3)

    return _mlp(g, Wm1, bm1.reshape(1, D), Wm2, bm2.reshape(1, 1))


# speedup vs baseline: 1.3349x; 1.1208x over previous
import functools

import jax
import jax.numpy as jnp
from jax import lax
from jax.experimental import pallas as pl
from jax.experimental.pallas import tpu as pltpu
from jax.experimental.pallas import tpu_sc as plsc

N = 10000
E = 320000
D = 128
G = 256

NC = 2
NS = 16
NW = NC * NS

CHUNK = 128
NCHUNKS = 2560
E_PAD = NCHUNKS * CHUNK
CPW0 = 144
CPW1 = 16
N_PAD = 10240
ACC_ROWS = 10112
ROWS_PER_TILE = ACC_ROWS // NS

BN = 1024
NB = N_PAD // BN


NBUF = 2


def _sc_agg_body(h_hbm, idx_hbm, zeros_hbm, out_hbm,
                 idx_v, src_u, dst_u, rows, acc, gs, ss):
    cc = lax.axis_index("c")
    s = lax.axis_index("s")

    r0 = s * ROWS_PER_TILE
    pltpu.sync_copy(zeros_hbm.at[pl.ds(r0, ROWS_PER_TILE)],
                    acc.at[pl.ds(r0, ROWS_PER_TILE)])

    @pl.when(cc == 0)
    def _():
        base = s * CPW0
        pltpu.sync_copy(idx_hbm.at[pl.ds(base, CPW0)], idx_v)

    @pl.when(cc == 1)
    def _():
        base = NS * CPW0 + s * CPW1
        pltpu.sync_copy(idx_hbm.at[pl.ds(base, CPW1)],
                        idx_v.at[pl.ds(0, CPW1)])

    plsc.subcore_barrier()

    def step(c):
        for q in range(CHUNK // 16):
            v = idx_v[c, pl.ds(q * 16, 16)]
            src_u[0, pl.ds(q * 16, 16)] = v & 0xFFFF
            dst_u[0, pl.ds(q * 16, 16)] = lax.shift_right_logical(v, 16)
        pltpu.async_copy(h_hbm.at[src_u.at[0]], rows.at[0], gs.at[0])
        pltpu.make_async_copy(h_hbm.at[src_u.at[0]], rows.at[0],
                              gs.at[0]).wait()
        pltpu.async_copy(rows.at[0], acc.at[dst_u.at[0]], ss.at[0], add=True)
        pltpu.make_async_copy(rows.at[0], acc.at[dst_u.at[0]],
                              ss.at[0]).wait()

    @pl.when(cc == 0)
    def _():
        @pl.loop(0, CPW0, step=1)
        def _(c):
            step(c)

    @pl.when(cc == 1)
    def _():
        @pl.loop(0, CPW1, step=1)
        def _(c):
            step(c)

    plsc.subcore_barrier()

    pltpu.sync_copy(acc.at[pl.ds(r0, ROWS_PER_TILE)],
                    out_hbm.at[cc, pl.ds(r0, ROWS_PER_TILE)])


_sc_agg = functools.partial(
    pl.kernel,
    out_type=jax.ShapeDtypeStruct((NC, N_PAD, D), jnp.float32),
    mesh=plsc.VectorSubcoreMesh(core_axis_name="c", subcore_axis_name="s"),
    scratch_types=[
        pltpu.VMEM((CPW0, CHUNK), jnp.int32),
        pltpu.VMEM((1, CHUNK), jnp.int32),
        pltpu.VMEM((1, CHUNK), jnp.int32),
        pltpu.VMEM((1, CHUNK, D), jnp.float32),
        pltpu.VMEM_SHARED((ACC_ROWS, D), jnp.float32),
        pltpu.SemaphoreType.DMA((1,)),
        pltpu.SemaphoreType.DMA((1,)),
    ],
)(_sc_agg_body)


def _dense_body(p_ref, h_ref, w_ref, r_ref, b_ref, rb_ref, o_ref):
    agg = p_ref[0] + p_ref[1]
    conv = jnp.maximum(
        jnp.dot(agg, w_ref[...], preferred_element_type=jnp.float32,
                precision=lax.Precision.HIGHEST)
        + b_ref[...], 0.0)
    res = jnp.maximum(
        jnp.dot(h_ref[...], r_ref[...], preferred_element_type=jnp.float32,
                precision=lax.Precision.HIGHEST)
        + rb_ref[...], 0.0)
    o_ref[...] = conv + res


_dense = pl.pallas_call(
    _dense_body,
    grid=(NB,),
    in_specs=[
        pl.BlockSpec((2, BN, D), lambda i: (0, i, 0)),
        pl.BlockSpec((BN, D), lambda i: (i, 0)),
        pl.BlockSpec((D, D), lambda i: (0, 0)),
        pl.BlockSpec((D, D), lambda i: (0, 0)),
        pl.BlockSpec((1, D), lambda i: (0, 0)),
        pl.BlockSpec((1, D), lambda i: (0, 0)),
    ],
    out_specs=pl.BlockSpec((BN, D), lambda i: (i, 0)),
    out_shape=jax.ShapeDtypeStruct((N_PAD, D), jnp.float32),
)


def _dense_pool_body(p_ref, h_ref, w_ref, r_ref, b_ref, rb_ref, n2g_ref,
                     o_ref, g_ref):
    agg = p_ref[0] + p_ref[1]
    conv = jnp.maximum(
        jnp.dot(agg, w_ref[...], preferred_element_type=jnp.float32,
                precision=lax.Precision.HIGHEST)
        + b_ref[...], 0.0)
    res = jnp.maximum(
        jnp.dot(h_ref[...], r_ref[...], preferred_element_type=jnp.float32,
                precision=lax.Precision.HIGHEST)
        + rb_ref[...], 0.0)
    hn = conv + res
    o_ref[...] = hn

    n2g = n2g_ref[0, 0]
    hn = jnp.where(n2g[:, None] < G, hn, 0.0)
    onehot = (n2g[:, None]
              == lax.broadcasted_iota(jnp.int32, (BN, G), 1)).astype(jnp.float32)
    gpart = lax.dot_general(onehot, hn, (((0,), (0,)), ((), ())),
                            preferred_element_type=jnp.float32,
                precision=lax.Precision.HIGHEST)

    @pl.when(pl.program_id(0) == 0)
    def _():
        g_ref[...] = jnp.zeros_like(g_ref)

    g_ref[...] += gpart


_dense_pool = pl.pallas_call(
    _dense_pool_body,
    grid=(NB,),
    in_specs=[
        pl.BlockSpec((2, BN, D), lambda i: (0, i, 0)),
        pl.BlockSpec((BN, D), lambda i: (i, 0)),
        pl.BlockSpec((D, D), lambda i: (0, 0)),
        pl.BlockSpec((D, D), lambda i: (0, 0)),
        pl.BlockSpec((1, D), lambda i: (0, 0)),
        pl.BlockSpec((1, D), lambda i: (0, 0)),
        pl.BlockSpec((1, 1, BN), lambda i: (i, 0, 0)),
    ],
    out_specs=[
        pl.BlockSpec((BN, D), lambda i: (i, 0)),
        pl.BlockSpec((G, D), lambda i: (0, 0)),
    ],
    out_shape=[
        jax.ShapeDtypeStruct((N_PAD, D), jnp.float32),
        jax.ShapeDtypeStruct((G, D), jnp.float32),
    ],
)


def _mlp_body(g_ref, w1_ref, b1_ref, w2_ref, b2_ref, o_ref):
    hm = jnp.maximum(
        jnp.dot(g_ref[...], w1_ref[...], preferred_element_type=jnp.float32,
                precision=lax.Precision.HIGHEST)
        + b1_ref[...], 0.0)
    o_ref[...] = (jnp.dot(hm, w2_ref[...], preferred_element_type=jnp.float32,
                precision=lax.Precision.HIGHEST)
                  + b2_ref[...])


_mlp = pl.pallas_call(
    _mlp_body,
    out_shape=jax.ShapeDtypeStruct((G, 1), jnp.float32),
)


@jax.jit
def kernel(graph_feats, edge_index, node2graph,
           W1, b1, R1, rb1, W2, b2, R2, rb2, W3, b3, R3, rb3,
           Wm1, bm1, Wm2, bm2):
    src = edge_index[0]
    dst = edge_index[1]
    pad = E_PAD - E
    src_p = jnp.concatenate([src, jnp.zeros((pad,), jnp.int32)])
    dst_p = jnp.concatenate([dst, jnp.full((pad,), N, jnp.int32)])
    idx_r = (src_p | (dst_p << 16)).reshape(NCHUNKS, CHUNK)

    h = jnp.pad(graph_feats, ((0, N_PAD - N), (0, 0)))
    zeros = jnp.zeros((N_PAD, D), jnp.float32)
    n2g3 = jnp.pad(node2graph, (0, N_PAD - N),
                   constant_values=G).reshape(NB, 1, BN)

    for (W, b, Rw, rb) in ((W1, b1, R1, rb1), (W2, b2, R2, rb2)):
        parts = _sc_agg(h, idx_r, zeros)
        h = _dense(parts, h, W, Rw, b.reshape(1, D), rb.reshape(1, D))

    parts = _sc_agg(h, idx_r, zeros)
    h, g = _dense_pool(parts, h, W3, R3, b3.reshape(1, D), rb3.reshape(1, D),
                       n2g3)

    return _mlp(g, Wm1, bm1.reshape(1, D), Wm2, bm2.reshape(1, 1))
